# Initial kernel scaffold; baseline (speedup 1.0000x reference)
import jax
import jax.numpy as jnp
from jax.experimental import pallas as pl


def kernel(feature, A, W1, b1, W2, b2, weight_b):
    raise NotImplementedError("write your pallas kernel here")


# speedup vs baseline: 3.9493x; 3.9493x over previous
import jax
import jax.numpy as jnp
from jax.experimental import pallas as pl
from jax.experimental.pallas import tpu as pltpu


def _mm_f32_kernel(x_ref, w_ref, o_ref):
    o_ref[...] = jnp.dot(x_ref[...], w_ref[...], preferred_element_type=jnp.float32)


def _mm_bf16_kernel(x_ref, w_ref, o_ref):
    o_ref[...] = jnp.dot(
        x_ref[...], w_ref[...], preferred_element_type=jnp.float32
    ).astype(jnp.bfloat16)


def _pass1_kernel(wb_ref, a_ref, s1_ref, b1_ref, u1_ref, abf_ref, macc_ref):
    v = pl.program_id(1)
    w = wb_ref[...]
    wv = jnp.where(v == 0, w[0, 0], jnp.where(v == 1, w[1, 0], w[2, 0]))
    contrib = wv * a_ref[0]

    @pl.when(v == 0)
    def _():
        macc_ref[...] = contrib

    @pl.when(v > 0)
    def _():
        macc_ref[...] += contrib

    @pl.when(v == 2)
    def _():
        m = macc_ref[...]
        abf_ref[...] = m.astype(jnp.bfloat16)
        u1_ref[...] = (
            jnp.dot(m, s1_ref[...], preferred_element_type=jnp.float32)
            + b1_ref[...]
        )


def _pass2_kernel(abf_ref, s2_ref, u1_ref, b2_ref, o_ref):
    o_ref[...] = (
        jnp.dot(abf_ref[...], s2_ref[...], preferred_element_type=jnp.float32)
        + u1_ref[...]
        + b2_ref[...]
    ) * 0.5


def kernel(feature, A, W1, b1, W2, b2, weight_b):
    n, f = feature.shape
    out = W1.shape[1]
    bm = 80 if n % 80 == 0 else n
    gi = n // bm
    bm2 = 400 if n % 400 == 0 else n
    gi2 = n // bm2

    b1r = b1.reshape(1, out)
    b2r = b2.reshape(1, out)

    support1 = pl.pallas_call(
        _mm_f32_kernel,
        out_shape=jax.ShapeDtypeStruct((n, out), jnp.float32),
    )(feature, W1)

    u1, a_bf = pl.pallas_call(
        _pass1_kernel,
        grid=(gi, 3),
        in_specs=[
            pl.BlockSpec((3, 1), lambda i, v: (0, 0)),
            pl.BlockSpec((1, bm, n), lambda i, v: (v, i, 0)),
            pl.BlockSpec((n, out), lambda i, v: (0, 0)),
            pl.BlockSpec((1, out), lambda i, v: (0, 0)),
        ],
        out_specs=[
            pl.BlockSpec((bm, out), lambda i, v: (i, 0)),
            pl.BlockSpec((bm, n), lambda i, v: (i, 0)),
        ],
        out_shape=[
            jax.ShapeDtypeStruct((n, out), jnp.float32),
            jax.ShapeDtypeStruct((n, n), jnp.bfloat16),
        ],
        scratch_shapes=[pltpu.VMEM((bm, n), jnp.float32)],
        compiler_params=pltpu.CompilerParams(
            dimension_semantics=("parallel", "arbitrary"),
        ),
    )(weight_b, A, support1, b1r)

    support2 = pl.pallas_call(
        _mm_bf16_kernel,
        out_shape=jax.ShapeDtypeStruct((n, out), jnp.bfloat16),
    )(u1, W2)

    result = pl.pallas_call(
        _pass2_kernel,
        grid=(gi2,),
        in_specs=[
            pl.BlockSpec((bm2, n), lambda i: (i, 0)),
            pl.BlockSpec((n, out), lambda i: (0, 0)),
            pl.BlockSpec((bm2, out), lambda i: (i, 0)),
            pl.BlockSpec((1, out), lambda i: (0, 0)),
        ],
        out_specs=pl.BlockSpec((bm2, out), lambda i: (i, 0)),
        out_shape=jax.ShapeDtypeStruct((n, out), jnp.float32),
        compiler_params=pltpu.CompilerParams(
            dimension_semantics=("parallel",),
        ),
    )(a_bf, support2, u1, b2r)

    return result


# speedup vs baseline: 3.9623x; 1.0033x over previous
import jax
import jax.numpy as jnp
from jax.experimental import pallas as pl
from jax.experimental.pallas import tpu as pltpu


def _mm_f32_kernel(x_ref, w_ref, o_ref):
    o_ref[...] = jnp.dot(x_ref[...], w_ref[...], preferred_element_type=jnp.float32)


def _mm_bf16_kernel(x_ref, w_ref, o_ref):
    o_ref[...] = jnp.dot(
        x_ref[...], w_ref[...], preferred_element_type=jnp.float32
    ).astype(jnp.bfloat16)


def _pass1_kernel(wb_ref, a_ref, s1_ref, b1_ref, u1_ref, abf_ref, macc_ref):
    v = pl.program_id(1)
    w = wb_ref[...]
    wv = jnp.where(v == 0, w[0, 0], jnp.where(v == 1, w[1, 0], w[2, 0]))
    contrib = wv * a_ref[0]

    @pl.when(v == 0)
    def _():
        macc_ref[...] = contrib

    @pl.when(v > 0)
    def _():
        macc_ref[...] += contrib

    @pl.when(v == 2)
    def _():
        mb = macc_ref[...].astype(jnp.bfloat16)
        abf_ref[...] = mb
        u1_ref[...] = (
            jnp.dot(mb, s1_ref[...], preferred_element_type=jnp.float32)
            + b1_ref[...]
        )


def _pass2_kernel(abf_ref, s2_ref, u1_ref, b2_ref, o_ref):
    o_ref[...] = (
        jnp.dot(abf_ref[...], s2_ref[...], preferred_element_type=jnp.float32)
        + u1_ref[...]
        + b2_ref[...]
    ) * 0.5


def kernel(feature, A, W1, b1, W2, b2, weight_b):
    n, f = feature.shape
    out = W1.shape[1]
    bm = 80 if n % 80 == 0 else n
    gi = n // bm
    bm2 = 400 if n % 400 == 0 else n
    gi2 = n // bm2

    b1r = b1.reshape(1, out)
    b2r = b2.reshape(1, out)

    support1 = pl.pallas_call(
        _mm_bf16_kernel,
        out_shape=jax.ShapeDtypeStruct((n, out), jnp.bfloat16),
    )(feature, W1)

    u1, a_bf = pl.pallas_call(
        _pass1_kernel,
        grid=(gi, 3),
        in_specs=[
            pl.BlockSpec((3, 1), lambda i, v: (0, 0)),
            pl.BlockSpec((1, bm, n), lambda i, v: (v, i, 0)),
            pl.BlockSpec((n, out), lambda i, v: (0, 0)),
            pl.BlockSpec((1, out), lambda i, v: (0, 0)),
        ],
        out_specs=[
            pl.BlockSpec((bm, out), lambda i, v: (i, 0)),
            pl.BlockSpec((bm, n), lambda i, v: (i, 0)),
        ],
        out_shape=[
            jax.ShapeDtypeStruct((n, out), jnp.float32),
            jax.ShapeDtypeStruct((n, n), jnp.bfloat16),
        ],
        scratch_shapes=[pltpu.VMEM((bm, n), jnp.float32)],
        compiler_params=pltpu.CompilerParams(
            dimension_semantics=("parallel", "arbitrary"),
        ),
    )(weight_b, A, support1, b1r)

    support2 = pl.pallas_call(
        _mm_bf16_kernel,
        out_shape=jax.ShapeDtypeStruct((n, out), jnp.bfloat16),
    )(u1, W2)

    result = pl.pallas_call(
        _pass2_kernel,
        grid=(gi2,),
        in_specs=[
            pl.BlockSpec((bm2, n), lambda i: (i, 0)),
            pl.BlockSpec((n, out), lambda i: (0, 0)),
            pl.BlockSpec((bm2, out), lambda i: (i, 0)),
            pl.BlockSpec((1, out), lambda i: (0, 0)),
        ],
        out_specs=pl.BlockSpec((bm2, out), lambda i: (i, 0)),
        out_shape=jax.ShapeDtypeStruct((n, out), jnp.float32),
        compiler_params=pltpu.CompilerParams(
            dimension_semantics=("parallel",),
        ),
    )(a_bf, support2, u1, b2r)

    return result


# speedup vs baseline: 4.7100x; 1.1887x over previous
import jax
import jax.numpy as jnp
from jax.experimental import pallas as pl
from jax.experimental.pallas import tpu as pltpu


def _mm_f32_kernel(x_ref, w_ref, o_ref):
    o_ref[...] = jnp.dot(x_ref[...], w_ref[...], preferred_element_type=jnp.float32)


def _mm_bf16_kernel(x_ref, w_ref, o_ref):
    o_ref[...] = jnp.dot(
        x_ref[...], w_ref[...], preferred_element_type=jnp.float32
    ).astype(jnp.bfloat16)


def _pass1_kernel(wb_ref, a_ref, s1_ref, b1_ref, u1_ref, abf_ref, macc_ref):
    v = pl.program_id(1)
    w = wb_ref[...]
    wv = jnp.where(v == 0, w[0, 0], jnp.where(v == 1, w[1, 0], w[2, 0]))
    contrib = wv * a_ref[0]

    @pl.when(v == 0)
    def _():
        macc_ref[...] = contrib

    @pl.when(v > 0)
    def _():
        macc_ref[...] += contrib

    @pl.when(v == 2)
    def _():
        mb = macc_ref[...].astype(jnp.bfloat16)
        abf_ref[...] = mb
        u1_ref[...] = (
            jnp.dot(mb, s1_ref[...], preferred_element_type=jnp.float32)
            + b1_ref[...]
        )


def _pass2_kernel(abf_ref, s2_ref, u1_ref, b2_ref, o_ref):
    o_ref[...] = (
        jnp.dot(abf_ref[...], s2_ref[...], preferred_element_type=jnp.float32)
        + u1_ref[...]
        + b2_ref[...]
    ) * 0.5


def kernel(feature, A, W1, b1, W2, b2, weight_b):
    n, f = feature.shape
    out = W1.shape[1]
    bm = 200 if n % 200 == 0 else n
    gi = n // bm
    bm2 = 400 if n % 400 == 0 else n
    gi2 = n // bm2

    b1r = b1.reshape(1, out)
    b2r = b2.reshape(1, out)

    support1 = pl.pallas_call(
        _mm_bf16_kernel,
        out_shape=jax.ShapeDtypeStruct((n, out), jnp.bfloat16),
    )(feature, W1)

    u1, a_bf = pl.pallas_call(
        _pass1_kernel,
        grid=(gi, 3),
        in_specs=[
            pl.BlockSpec((3, 1), lambda i, v: (0, 0)),
            pl.BlockSpec((1, bm, n), lambda i, v: (v, i, 0)),
            pl.BlockSpec((n, out), lambda i, v: (0, 0)),
            pl.BlockSpec((1, out), lambda i, v: (0, 0)),
        ],
        out_specs=[
            pl.BlockSpec((bm, out), lambda i, v: (i, 0)),
            pl.BlockSpec((bm, n), lambda i, v: (i, 0)),
        ],
        out_shape=[
            jax.ShapeDtypeStruct((n, out), jnp.float32),
            jax.ShapeDtypeStruct((n, n), jnp.bfloat16),
        ],
        scratch_shapes=[pltpu.VMEM((bm, n), jnp.float32)],
        compiler_params=pltpu.CompilerParams(
            dimension_semantics=("parallel", "arbitrary"),
        ),
    )(weight_b, A, support1, b1r)

    support2 = pl.pallas_call(
        _mm_bf16_kernel,
        out_shape=jax.ShapeDtypeStruct((n, out), jnp.bfloat16),
    )(u1, W2)

    result = pl.pallas_call(
        _pass2_kernel,
        grid=(gi2,),
        in_specs=[
            pl.BlockSpec((bm2, n), lambda i: (i, 0)),
            pl.BlockSpec((n, out), lambda i: (0, 0)),
            pl.BlockSpec((bm2, out), lambda i: (i, 0)),
            pl.BlockSpec((1, out), lambda i: (0, 0)),
        ],
        out_specs=pl.BlockSpec((bm2, out), lambda i: (i, 0)),
        out_shape=jax.ShapeDtypeStruct((n, out), jnp.float32),
        compiler_params=pltpu.CompilerParams(
            dimension_semantics=("parallel",),
        ),
    )(a_bf, support2, u1, b2r)

    return result


# speedup vs baseline: 4.8957x; 1.0394x over previous
import jax
import jax.numpy as jnp
from jax.experimental import pallas as pl
from jax.experimental.pallas import tpu as pltpu


def _mm_f32_kernel(x_ref, w_ref, o_ref):
    o_ref[...] = jnp.dot(x_ref[...], w_ref[...], preferred_element_type=jnp.float32)


def _mm_bf16_kernel(x_ref, w_ref, o_ref):
    o_ref[...] = jnp.dot(
        x_ref[...], w_ref[...], preferred_element_type=jnp.float32
    ).astype(jnp.bfloat16)


def _pass1_kernel(wb_ref, a_ref, s1_ref, b1_ref, u1_ref, abf_ref):
    v = pl.program_id(1)
    w = wb_ref[...]
    wv = jnp.where(v == 0, w[0, 0], jnp.where(v == 1, w[1, 0], w[2, 0]))
    contrib = (wv * a_ref[0]).astype(jnp.bfloat16)

    @pl.when(v == 0)
    def _():
        abf_ref[...] = contrib

    @pl.when(v > 0)
    def _():
        abf_ref[...] += contrib

    @pl.when(v == 2)
    def _():
        u1_ref[...] = (
            jnp.dot(abf_ref[...], s1_ref[...], preferred_element_type=jnp.float32)
            + b1_ref[...]
        )


def _pass2_kernel(abf_ref, s2_ref, u1_ref, b2_ref, o_ref):
    o_ref[...] = (
        jnp.dot(abf_ref[...], s2_ref[...], preferred_element_type=jnp.float32)
        + u1_ref[...]
        + b2_ref[...]
    ) * 0.5


def kernel(feature, A, W1, b1, W2, b2, weight_b):
    n, f = feature.shape
    out = W1.shape[1]
    bm = 200 if n % 200 == 0 else n
    gi = n // bm
    bm2 = 1000 if n % 1000 == 0 else n
    gi2 = n // bm2

    b1r = b1.reshape(1, out)
    b2r = b2.reshape(1, out)

    support1 = pl.pallas_call(
        _mm_bf16_kernel,
        out_shape=jax.ShapeDtypeStruct((n, out), jnp.bfloat16),
    )(feature, W1)

    u1, a_bf = pl.pallas_call(
        _pass1_kernel,
        grid=(gi, 3),
        in_specs=[
            pl.BlockSpec((3, 1), lambda i, v: (0, 0)),
            pl.BlockSpec((1, bm, n), lambda i, v: (v, i, 0)),
            pl.BlockSpec((n, out), lambda i, v: (0, 0)),
            pl.BlockSpec((1, out), lambda i, v: (0, 0)),
        ],
        out_specs=[
            pl.BlockSpec((bm, out), lambda i, v: (i, 0)),
            pl.BlockSpec((bm, n), lambda i, v: (i, 0)),
        ],
        out_shape=[
            jax.ShapeDtypeStruct((n, out), jnp.float32),
            jax.ShapeDtypeStruct((n, n), jnp.bfloat16),
        ],
        compiler_params=pltpu.CompilerParams(
            dimension_semantics=("parallel", "arbitrary"),
        ),
    )(weight_b, A, support1, b1r)

    support2 = pl.pallas_call(
        _mm_bf16_kernel,
        out_shape=jax.ShapeDtypeStruct((n, out), jnp.bfloat16),
    )(u1, W2)

    result = pl.pallas_call(
        _pass2_kernel,
        grid=(gi2,),
        in_specs=[
            pl.BlockSpec((bm2, n), lambda i: (i, 0)),
            pl.BlockSpec((n, out), lambda i: (0, 0)),
            pl.BlockSpec((bm2, out), lambda i: (i, 0)),
            pl.BlockSpec((1, out), lambda i: (0, 0)),
        ],
        out_specs=pl.BlockSpec((bm2, out), lambda i: (i, 0)),
        out_shape=jax.ShapeDtypeStruct((n, out), jnp.float32),
        compiler_params=pltpu.CompilerParams(
            dimension_semantics=("parallel",),
        ),
    )(a_bf, support2, u1, b2r)

    return result
